# initial kernel scaffold (unmeasured)
import jax
import jax.numpy as jnp
from jax import lax
from jax.experimental import pallas as pl
from jax.experimental.pallas import tpu as pltpu

N_DEV = 8


def kernel(x, w_mat, scale_x, scale_w):
    m_per, k = x.shape
    n_total = w_mat.shape[1]
    n_per = n_total // N_DEV

    my = lax.axis_index("i")
    w_loc = lax.dynamic_slice(w_mat, (0, my * n_per), (k, n_per))
    scale = (scale_x * scale_w).astype(jnp.float32)

    def body(x_ref, w_ref, s_ref, out_ref, comm_ref, send_sems, recv_sems):
        me = lax.axis_index("i")
        left = lax.rem(me + N_DEV - 1, N_DEV)
        right = lax.rem(me + 1, N_DEV)
        s = s_ref[0]

        barrier_sem = pltpu.get_barrier_semaphore()
        for nbr in (left, right):
            pl.semaphore_signal(
                barrier_sem, inc=1,
                device_id=(nbr,), device_id_type=pl.DeviceIdType.MESH,
            )
        pl.semaphore_wait(barrier_sem, 2)

        def chunk_gemm(src_ref, origin):
            acc = jnp.dot(src_ref[...], w_ref[...],
                          preferred_element_type=jnp.float32)
            out_ref[pl.ds(origin * m_per, m_per), :] = jnp.maximum(acc * s, 0.0)

        def hop_rdma(h):
            return pltpu.make_async_remote_copy(
                src_ref=comm_ref.at[h],
                dst_ref=comm_ref.at[h + 1],
                send_sem=send_sems.at[h],
                recv_sem=recv_sems.at[h],
                device_id=(right,),
                device_id_type=pl.DeviceIdType.MESH,
            )

        comm_ref[0] = x_ref[...]
        hop_rdma(0).start()
        chunk_gemm(x_ref, me)

        for h in range(N_DEV - 1):
            hop_rdma(h).wait_recv()
            if h + 1 < N_DEV - 1:
                hop_rdma(h + 1).start()
            origin = lax.rem(me + (N_DEV - 1 - h), N_DEV)
            chunk_gemm(comm_ref.at[h + 1], origin)

        for h in range(N_DEV - 1):
            hop_rdma(h).wait_send()

    return pl.pallas_call(
        body,
        out_shape=jax.ShapeDtypeStruct((N_DEV * m_per, n_per), jnp.float32),
        in_specs=[
            pl.BlockSpec(memory_space=pltpu.VMEM),
            pl.BlockSpec(memory_space=pltpu.VMEM),
            pl.BlockSpec(memory_space=pltpu.SMEM),
        ],
        out_specs=pl.BlockSpec(memory_space=pltpu.VMEM),
        scratch_shapes=[
            pltpu.VMEM((N_DEV, m_per, k), x.dtype),
            pltpu.SemaphoreType.DMA((N_DEV - 1,)),
            pltpu.SemaphoreType.DMA((N_DEV - 1,)),
        ],
        compiler_params=pltpu.CompilerParams(collective_id=0),
    )(x, w_loc, scale)


# baseline (device time: 207912 ns/iter reference)
import jax
import jax.numpy as jnp
from jax import lax
from jax.experimental import pallas as pl
from jax.experimental.pallas import tpu as pltpu

N_DEV = 8


def kernel(x, w_mat, scale_x, scale_w):
    m_per, k = x.shape
    n_total = w_mat.shape[1]
    n_per = n_total // N_DEV

    my = lax.axis_index("i")
    w_loc = lax.dynamic_slice(w_mat, (0, my * n_per), (k, n_per))
    x = x.astype(jnp.float8_e4m3fn)
    w_loc = w_loc.astype(jnp.float8_e4m3fn)
    scale = (scale_x * scale_w).astype(jnp.float32)

    def body(x_ref, w_ref, s_ref, out_ref, comm_ref, send_sems, recv_sems):
        me = lax.axis_index("i")
        left = lax.rem(me + N_DEV - 1, N_DEV)
        right = lax.rem(me + 1, N_DEV)
        s = s_ref[0]

        barrier_sem = pltpu.get_barrier_semaphore()
        for nbr in (left, right):
            pl.semaphore_signal(
                barrier_sem, inc=1,
                device_id=(nbr,), device_id_type=pl.DeviceIdType.MESH,
            )
        pl.semaphore_wait(barrier_sem, 2)

        def chunk_gemm(src_ref, origin):
            acc = jnp.dot(src_ref[...], w_ref[...],
                          preferred_element_type=jnp.float32)
            out_ref[pl.ds(origin * m_per, m_per), :] = jnp.maximum(acc * s, 0.0)

        def hop_rdma(h):
            return pltpu.make_async_remote_copy(
                src_ref=comm_ref.at[h],
                dst_ref=comm_ref.at[h + 1],
                send_sem=send_sems.at[h],
                recv_sem=recv_sems.at[h],
                device_id=(right,),
                device_id_type=pl.DeviceIdType.MESH,
            )

        comm_ref[0] = x_ref[...]
        hop_rdma(0).start()
        chunk_gemm(x_ref, me)

        for h in range(N_DEV - 1):
            hop_rdma(h).wait_recv()
            if h + 1 < N_DEV - 1:
                hop_rdma(h + 1).start()
            origin = lax.rem(me + (N_DEV - 1 - h), N_DEV)
            chunk_gemm(comm_ref.at[h + 1], origin)

        for h in range(N_DEV - 1):
            hop_rdma(h).wait_send()

    return pl.pallas_call(
        body,
        out_shape=jax.ShapeDtypeStruct((N_DEV * m_per, n_per), jnp.float32),
        in_specs=[
            pl.BlockSpec(memory_space=pltpu.VMEM),
            pl.BlockSpec(memory_space=pltpu.VMEM),
            pl.BlockSpec(memory_space=pltpu.SMEM),
        ],
        out_specs=pl.BlockSpec(memory_space=pltpu.VMEM),
        scratch_shapes=[
            pltpu.VMEM((N_DEV, m_per, k), x.dtype),
            pltpu.SemaphoreType.DMA((N_DEV - 1,)),
            pltpu.SemaphoreType.DMA((N_DEV - 1,)),
        ],
        compiler_params=pltpu.CompilerParams(collective_id=0),
    )(x, w_loc, scale)


# device time: 148697 ns/iter; 1.3982x vs baseline; 1.3982x over previous
import jax
import jax.numpy as jnp
from jax import lax
from jax.experimental import pallas as pl
from jax.experimental.pallas import tpu as pltpu

N_DEV = 8
N_CW = 4
N_CCW = 3

_ORDER = (0, 1, 2, 3, 7, 6, 5, 4)
_NEXT = [0] * N_DEV
_PREV = [0] * N_DEV
for _i, _p in enumerate(_ORDER):
    _NEXT[_p] = _ORDER[(_i + 1) % N_DEV]
    _PREV[_p] = _ORDER[(_i - 1) % N_DEV]


def kernel(x, w_mat, scale_x, scale_w):
    m_per, k = x.shape
    n_total = w_mat.shape[1]
    n_per = n_total // N_DEV

    my = lax.axis_index("i")
    w_loc = lax.dynamic_slice(w_mat, (0, my * n_per), (k, n_per))
    x = x.astype(jnp.float8_e4m3fn)
    w_loc = w_loc.astype(jnp.float8_e4m3fn)
    scale = (scale_x * scale_w).astype(jnp.float32)

    nxt = jnp.asarray(_NEXT, jnp.int32)
    prv = jnp.asarray(_PREV, jnp.int32)
    walk = []
    p = my.astype(jnp.int32)
    for _ in range(N_CW):
        p = prv[p]
        walk.append(p)
    p = my.astype(jnp.int32)
    for _ in range(N_CCW):
        p = nxt[p]
        walk.append(p)
    nbrs = jnp.stack(walk)

    def body(x_ref, w_ref, s_ref, nbr_ref, out_ref,
             cw_ref, ccw_ref, cw_send, cw_recv, ccw_send, ccw_recv):
        s = s_ref[0]
        right = nbr_ref[N_CW]
        left = nbr_ref[0]

        barrier_sem = pltpu.get_barrier_semaphore()
        for nbr in (left, right):
            pl.semaphore_signal(
                barrier_sem, inc=1,
                device_id=(nbr,), device_id_type=pl.DeviceIdType.MESH,
            )
        pl.semaphore_wait(barrier_sem, 2)

        def chunk_gemm(src_ref, origin):
            acc = jnp.dot(src_ref[...], w_ref[...],
                          preferred_element_type=jnp.float32)
            out_ref[pl.ds(origin * m_per, m_per), :] = jnp.maximum(acc * s, 0.0)

        def cw_rdma(h):
            return pltpu.make_async_remote_copy(
                src_ref=x_ref if h == 0 else cw_ref.at[h - 1],
                dst_ref=cw_ref.at[h],
                send_sem=cw_send.at[h],
                recv_sem=cw_recv.at[h],
                device_id=(right,),
                device_id_type=pl.DeviceIdType.MESH,
            )

        def ccw_rdma(h):
            return pltpu.make_async_remote_copy(
                src_ref=x_ref if h == 0 else ccw_ref.at[h - 1],
                dst_ref=ccw_ref.at[h],
                send_sem=ccw_send.at[h],
                recv_sem=ccw_recv.at[h],
                device_id=(left,),
                device_id_type=pl.DeviceIdType.MESH,
            )

        cw_rdma(0).start()
        ccw_rdma(0).start()
        my_pos = lax.axis_index("i")
        chunk_gemm(x_ref, my_pos)

        for h in range(N_CW):
            cw_rdma(h).wait_recv()
            if h + 1 < N_CW:
                cw_rdma(h + 1).start()
            if h < N_CCW:
                ccw_rdma(h).wait_recv()
                if h + 1 < N_CCW:
                    ccw_rdma(h + 1).start()
            chunk_gemm(cw_ref.at[h], nbr_ref[h])
            if h < N_CCW:
                chunk_gemm(ccw_ref.at[h], nbr_ref[N_CW + h])

        for h in range(N_CW):
            cw_rdma(h).wait_send()
        for h in range(N_CCW):
            ccw_rdma(h).wait_send()

    return pl.pallas_call(
        body,
        out_shape=jax.ShapeDtypeStruct((N_DEV * m_per, n_per), jnp.float32),
        in_specs=[
            pl.BlockSpec(memory_space=pltpu.VMEM),
            pl.BlockSpec(memory_space=pltpu.VMEM),
            pl.BlockSpec(memory_space=pltpu.SMEM),
            pl.BlockSpec(memory_space=pltpu.SMEM),
        ],
        out_specs=pl.BlockSpec(memory_space=pltpu.VMEM),
        scratch_shapes=[
            pltpu.VMEM((N_CW, m_per, k), jnp.float8_e4m3fn),
            pltpu.VMEM((N_CCW, m_per, k), jnp.float8_e4m3fn),
            pltpu.SemaphoreType.DMA((N_CW,)),
            pltpu.SemaphoreType.DMA((N_CW,)),
            pltpu.SemaphoreType.DMA((N_CCW,)),
            pltpu.SemaphoreType.DMA((N_CCW,)),
        ],
        compiler_params=pltpu.CompilerParams(collective_id=0),
    )(x, w_loc, scale, nbrs)


# device time: 140051 ns/iter; 1.4845x vs baseline; 1.0617x over previous
import jax
import jax.numpy as jnp
from jax import lax
from jax.experimental import pallas as pl
from jax.experimental.pallas import tpu as pltpu

N_DEV = 8
N_HOP = 4
W_PIECES = 4

_ORDER = (0, 1, 2, 3, 7, 6, 5, 4)
_NEXT = [0] * N_DEV
_PREV = [0] * N_DEV
for _i, _p in enumerate(_ORDER):
    _NEXT[_p] = _ORDER[(_i + 1) % N_DEV]
    _PREV[_p] = _ORDER[(_i - 1) % N_DEV]


def kernel(x, w_mat, scale_x, scale_w):
    m_per, k = x.shape
    n_total = w_mat.shape[1]
    n_per = n_total // N_DEV
    m_half = m_per // 2
    k_piece = k // W_PIECES

    my = lax.axis_index("i")
    scale = (scale_x * scale_w).astype(jnp.float32)

    nxt = jnp.asarray(_NEXT, jnp.int32)
    prv = jnp.asarray(_PREV, jnp.int32)
    walk = []
    p = my.astype(jnp.int32)
    for _ in range(4):
        p = prv[p]
        walk.append(p)
    p = my.astype(jnp.int32)
    for _ in range(3):
        p = nxt[p]
        walk.append(p)
    walk.append(my.astype(jnp.int32) * n_per)
    nbrs = jnp.stack(walk)

    def body(x_ref, w_hbm, s_ref, nbr_ref, out_ref,
             x8_ref, w8_ref, wstage_ref, cw_ref, ccw_ref,
             wdma_sems, cw_send, cw_recv, ccw_send, ccw_recv):
        s = s_ref[0]
        right = nbr_ref[4]
        left = nbr_ref[0]
        col0 = pl.multiple_of(nbr_ref[7], n_per)

        x8_ref[...] = x_ref[...].astype(jnp.float8_e4m3fn)

        def w_dma(j):
            return pltpu.make_async_copy(
                w_hbm.at[pl.ds(j * k_piece, k_piece), pl.ds(col0, n_per)],
                wstage_ref.at[j % 2],
                wdma_sems.at[j % 2],
            )

        w_dma(0).start()

        barrier_sem = pltpu.get_barrier_semaphore()
        for nbr in (left, right):
            pl.semaphore_signal(
                barrier_sem, inc=1,
                device_id=(nbr,), device_id_type=pl.DeviceIdType.MESH,
            )
        pl.semaphore_wait(barrier_sem, 2)

        def cw_rdma(h):
            src = x8_ref if h == 0 else cw_ref.at[h - 1]
            return pltpu.make_async_remote_copy(
                src_ref=src, dst_ref=cw_ref.at[h],
                send_sem=cw_send.at[h], recv_sem=cw_recv.at[h],
                device_id=(right,), device_id_type=pl.DeviceIdType.MESH,
            )

        def ccw_rdma(h):
            src = x8_ref if h == 0 else ccw_ref.at[h - 1]
            return pltpu.make_async_remote_copy(
                src_ref=src, dst_ref=ccw_ref.at[h],
                send_sem=ccw_send.at[h], recv_sem=ccw_recv.at[h],
                device_id=(left,), device_id_type=pl.DeviceIdType.MESH,
            )

        cw_rdma(0).start()
        ccw_rdma(0).start()

        for j in range(W_PIECES):
            if j + 1 < W_PIECES:
                w_dma(j + 1).start()
            w_dma(j).wait()
            w8_ref[pl.ds(j * k_piece, k_piece), :] = (
                wstage_ref[j % 2].astype(jnp.float8_e4m3fn))

        def chunk_gemm(src_ref, origin):
            acc = jnp.dot(src_ref[...], w8_ref[...],
                          preferred_element_type=jnp.float32)
            out_ref[pl.ds(origin * m_per, m_per), :] = jnp.maximum(acc * s, 0.0)

        chunk_gemm(x8_ref, lax.axis_index("i"))

        for h in range(3):
            cw_rdma(h).wait_recv()
            cw_rdma(h + 1).start()
            if h < 2:
                ccw_rdma(h).wait_recv()
                ccw_rdma(h + 1).start()
                chunk_gemm(ccw_ref.at[h], nbr_ref[4 + h])
            chunk_gemm(cw_ref.at[h], nbr_ref[h])

        ccw_rdma(2).wait_recv()
        chunk_gemm(ccw_ref.at[2], nbr_ref[6])
        cw_rdma(3).wait_recv()
        chunk_gemm(cw_ref.at[3], nbr_ref[3])

        for h in range(N_HOP):
            cw_rdma(h).wait_send()
            if h < 3:
                ccw_rdma(h).wait_send()

    f8 = jnp.float8_e4m3fn
    return pl.pallas_call(
        body,
        out_shape=jax.ShapeDtypeStruct((N_DEV * m_per, n_per), jnp.float32),
        in_specs=[
            pl.BlockSpec(memory_space=pltpu.VMEM),
            pl.BlockSpec(memory_space=pl.ANY),
            pl.BlockSpec(memory_space=pltpu.SMEM),
            pl.BlockSpec(memory_space=pltpu.SMEM),
        ],
        out_specs=pl.BlockSpec(memory_space=pltpu.VMEM),
        scratch_shapes=[
            pltpu.VMEM((m_per, k), f8),
            pltpu.VMEM((k, n_per), f8),
            pltpu.VMEM((2, k_piece, n_per), jnp.float32),
            pltpu.VMEM((N_HOP, m_per, k), f8),
            pltpu.VMEM((3, m_per, k), f8),
            pltpu.SemaphoreType.DMA((2,)),
            pltpu.SemaphoreType.DMA((N_HOP,)),
            pltpu.SemaphoreType.DMA((N_HOP,)),
            pltpu.SemaphoreType.DMA((N_HOP,)),
            pltpu.SemaphoreType.DMA((N_HOP,)),
        ],
        compiler_params=pltpu.CompilerParams(
            collective_id=0, vmem_limit_bytes=60 * 1024 * 1024),
    )(x, w_mat, scale, nbrs)


# device time: 126024 ns/iter; 1.6498x vs baseline; 1.1113x over previous
import jax
import jax.numpy as jnp
from jax import lax
from jax.experimental import pallas as pl
from jax.experimental.pallas import tpu as pltpu

N_DEV = 8
N_CW = 4
N_CCW = 3
W_PIECES = 4


def kernel(x, w_mat, scale_x, scale_w):
    m_per, k = x.shape
    n_total = w_mat.shape[1]
    n_per = n_total // N_DEV
    k_piece = k // W_PIECES

    def body(x_hbm, w_hbm, sx_ref, sw_ref, out_ref,
             x32_ref, x8_ref, w8_ref, wstage_ref, cw_ref, ccw_ref,
             xdma_sem, wdma_sems, cw_send, cw_recv, ccw_send, ccw_recv):
        me = lax.axis_index("i").astype(jnp.int32)
        s = sx_ref[0] * sw_ref[0]

        idx = jnp.where(me < 4, me, 11 - me)

        def pos_at(j):
            jm = lax.rem(j + 2 * N_DEV, N_DEV)
            return jnp.where(jm < 4, jm, 11 - jm)

        left = pos_at(idx - 1)
        right = pos_at(idx + 1)
        col0 = pl.multiple_of(me * n_per, n_per)

        x_dma = pltpu.make_async_copy(x_hbm, x32_ref, xdma_sem)
        x_dma.start()

        def w_dma(j):
            return pltpu.make_async_copy(
                w_hbm.at[pl.ds(j * k_piece, k_piece), pl.ds(col0, n_per)],
                wstage_ref.at[j % 2],
                wdma_sems.at[j % 2],
            )

        w_dma(0).start()
        x_dma.wait()
        x8_ref[...] = x32_ref[...].astype(jnp.float8_e4m3fn)

        barrier_sem = pltpu.get_barrier_semaphore()
        for nbr in (left, right):
            pl.semaphore_signal(
                barrier_sem, inc=1,
                device_id=(nbr,), device_id_type=pl.DeviceIdType.MESH,
            )
        pl.semaphore_wait(barrier_sem, 2)

        def cw_rdma(h):
            src = x8_ref if h == 0 else cw_ref.at[h - 1]
            return pltpu.make_async_remote_copy(
                src_ref=src, dst_ref=cw_ref.at[h],
                send_sem=cw_send.at[h], recv_sem=cw_recv.at[h],
                device_id=(right,), device_id_type=pl.DeviceIdType.MESH,
            )

        def ccw_rdma(h):
            src = x8_ref if h == 0 else ccw_ref.at[h - 1]
            return pltpu.make_async_remote_copy(
                src_ref=src, dst_ref=ccw_ref.at[h],
                send_sem=ccw_send.at[h], recv_sem=ccw_recv.at[h],
                device_id=(left,), device_id_type=pl.DeviceIdType.MESH,
            )

        cw_rdma(0).start()
        ccw_rdma(0).start()

        for j in range(W_PIECES):
            if j + 1 < W_PIECES:
                w_dma(j + 1).start()
            w_dma(j).wait()
            w8_ref[pl.ds(j * k_piece, k_piece), :] = (
                wstage_ref[j % 2].astype(jnp.float8_e4m3fn))

        def chunk_gemm(src_ref, origin):
            acc = jnp.dot(src_ref[...], w8_ref[...],
                          preferred_element_type=jnp.float32)
            out_ref[pl.ds(origin * m_per, m_per), :] = jnp.maximum(acc * s, 0.0)

        chunk_gemm(x8_ref, me)

        for h in range(3):
            cw_rdma(h).wait_recv()
            cw_rdma(h + 1).start()
            if h < 2:
                ccw_rdma(h).wait_recv()
                ccw_rdma(h + 1).start()
                chunk_gemm(ccw_ref.at[h], pos_at(idx + h + 1))
            chunk_gemm(cw_ref.at[h], pos_at(idx - h - 1))

        ccw_rdma(2).wait_recv()
        chunk_gemm(ccw_ref.at[2], pos_at(idx + 3))
        cw_rdma(3).wait_recv()
        chunk_gemm(cw_ref.at[3], pos_at(idx + 4))

        for h in range(N_CW):
            cw_rdma(h).wait_send()
            if h < N_CCW:
                ccw_rdma(h).wait_send()

    f8 = jnp.float8_e4m3fn
    return pl.pallas_call(
        body,
        out_shape=jax.ShapeDtypeStruct((N_DEV * m_per, n_per), jnp.float32),
        in_specs=[
            pl.BlockSpec(memory_space=pl.ANY),
            pl.BlockSpec(memory_space=pl.ANY),
            pl.BlockSpec(memory_space=pltpu.SMEM),
            pl.BlockSpec(memory_space=pltpu.SMEM),
        ],
        out_specs=pl.BlockSpec(memory_space=pltpu.VMEM),
        scratch_shapes=[
            pltpu.VMEM((m_per, k), jnp.float32),
            pltpu.VMEM((m_per, k), f8),
            pltpu.VMEM((k, n_per), f8),
            pltpu.VMEM((2, k_piece, n_per), jnp.float32),
            pltpu.VMEM((N_CW, m_per, k), f8),
            pltpu.VMEM((N_CCW, m_per, k), f8),
            pltpu.SemaphoreType.DMA,
            pltpu.SemaphoreType.DMA((2,)),
            pltpu.SemaphoreType.DMA((N_CW,)),
            pltpu.SemaphoreType.DMA((N_CW,)),
            pltpu.SemaphoreType.DMA((N_CCW,)),
            pltpu.SemaphoreType.DMA((N_CCW,)),
        ],
        compiler_params=pltpu.CompilerParams(
            collective_id=0, vmem_limit_bytes=60 * 1024 * 1024),
    )(x, w_mat, scale_x, scale_w)


# device time: 115001 ns/iter; 1.8079x vs baseline; 1.0959x over previous
import jax
import jax.numpy as jnp
from jax import lax
from jax.experimental import pallas as pl
from jax.experimental.pallas import tpu as pltpu

N_DEV = 8
N_CW = 4
N_CCW = 3
W_PIECES = 4


def kernel(x, w_mat, scale_x, scale_w):
    m_per, k = x.shape
    n_total = w_mat.shape[1]
    n_per = n_total // N_DEV
    k_piece = k // W_PIECES

    m_half = m_per // 2

    def body(x_hbm, w_hbm, sx_ref, sw_ref, out_ref,
             x32_ref, x8_ref, w8_ref, wstage_ref, cw_ref, ccw_ref,
             cwh_src, cwh_dst, ccwh_src, ccwh_dst,
             xdma_sem, wdma_sems, cw_send, cw_recv, ccw_send, ccw_recv):
        me = lax.axis_index("i").astype(jnp.int32)
        s = sx_ref[0] * sw_ref[0]

        idx = jnp.where(me < 4, me, 11 - me)

        def pos_at(j):
            jm = lax.rem(j + 2 * N_DEV, N_DEV)
            return jnp.where(jm < 4, jm, 11 - jm)

        left = pos_at(idx - 1)
        right = pos_at(idx + 1)
        col0 = pl.multiple_of(me * n_per, n_per)

        x_dma = pltpu.make_async_copy(x_hbm, x32_ref, xdma_sem)
        x_dma.start()

        def w_dma(j):
            return pltpu.make_async_copy(
                w_hbm.at[pl.ds(j * k_piece, k_piece), pl.ds(col0, n_per)],
                wstage_ref.at[j % 2],
                wdma_sems.at[j % 2],
            )

        w_dma(0).start()
        x_dma.wait()
        x8_ref[...] = x32_ref[...].astype(jnp.float8_e4m3fn)

        barrier_sem = pltpu.get_barrier_semaphore()
        for nbr in (left, right):
            pl.semaphore_signal(
                barrier_sem, inc=1,
                device_id=(nbr,), device_id_type=pl.DeviceIdType.MESH,
            )
        pl.semaphore_wait(barrier_sem, 2)

        def cw_rdma(h):
            if h == 3:
                src, dst = cwh_src, cwh_dst
            else:
                src = x8_ref if h == 0 else cw_ref.at[h - 1]
                dst = cw_ref.at[h]
            return pltpu.make_async_remote_copy(
                src_ref=src, dst_ref=dst,
                send_sem=cw_send.at[h], recv_sem=cw_recv.at[h],
                device_id=(right,), device_id_type=pl.DeviceIdType.MESH,
            )

        def ccw_rdma(h):
            if h == 3:
                src, dst = ccwh_src, ccwh_dst
            else:
                src = x8_ref if h == 0 else ccw_ref.at[h - 1]
                dst = ccw_ref.at[h]
            return pltpu.make_async_remote_copy(
                src_ref=src, dst_ref=dst,
                send_sem=ccw_send.at[h], recv_sem=ccw_recv.at[h],
                device_id=(left,), device_id_type=pl.DeviceIdType.MESH,
            )

        cw_rdma(0).start()
        ccw_rdma(0).start()

        for j in range(W_PIECES):
            if j + 1 < W_PIECES:
                w_dma(j + 1).start()
            w_dma(j).wait()
            w8_ref[pl.ds(j * k_piece, k_piece), :] = (
                wstage_ref[j % 2].astype(jnp.float8_e4m3fn))

        def chunk_gemm(src_ref, origin):
            acc = jnp.dot(src_ref[...], w8_ref[...],
                          preferred_element_type=jnp.float32)
            out_ref[pl.ds(origin * m_per, m_per), :] = jnp.maximum(acc * s, 0.0)

        chunk_gemm(x8_ref, me)

        for h in range(3):
            cw_rdma(h).wait_recv()
            if h < 2:
                cw_rdma(h + 1).start()
            else:
                cwh_src[...] = cw_ref[2, :m_half, :]
                cw_rdma(3).start()
            ccw_rdma(h).wait_recv()
            if h < 2:
                ccw_rdma(h + 1).start()
            else:
                ccwh_src[...] = ccw_ref[2, m_half:, :]
                ccw_rdma(3).start()
            chunk_gemm(cw_ref.at[h], pos_at(idx - h - 1))
            chunk_gemm(ccw_ref.at[h], pos_at(idx + h + 1))

        d4 = pos_at(idx + 4)
        cw_rdma(3).wait_recv()
        ccw_rdma(3).wait_recv()
        acc_top = jnp.dot(cwh_dst[...], w8_ref[...],
                          preferred_element_type=jnp.float32)
        out_ref[pl.ds(d4 * m_per, m_half), :] = jnp.maximum(acc_top * s, 0.0)
        acc_bot = jnp.dot(ccwh_dst[...], w8_ref[...],
                          preferred_element_type=jnp.float32)
        out_ref[pl.ds(d4 * m_per + m_half, m_half), :] = (
            jnp.maximum(acc_bot * s, 0.0))

        for h in range(N_CW):
            cw_rdma(h).wait_send()
            ccw_rdma(h).wait_send()

    f8 = jnp.float8_e4m3fn
    return pl.pallas_call(
        body,
        out_shape=jax.ShapeDtypeStruct((N_DEV * m_per, n_per), jnp.float32),
        in_specs=[
            pl.BlockSpec(memory_space=pl.ANY),
            pl.BlockSpec(memory_space=pl.ANY),
            pl.BlockSpec(memory_space=pltpu.SMEM),
            pl.BlockSpec(memory_space=pltpu.SMEM),
        ],
        out_specs=pl.BlockSpec(memory_space=pltpu.VMEM),
        scratch_shapes=[
            pltpu.VMEM((m_per, k), jnp.float32),
            pltpu.VMEM((m_per, k), f8),
            pltpu.VMEM((k, n_per), f8),
            pltpu.VMEM((2, k_piece, n_per), jnp.float32),
            pltpu.VMEM((3, m_per, k), f8),
            pltpu.VMEM((3, m_per, k), f8),
            pltpu.VMEM((m_per // 2, k), f8),
            pltpu.VMEM((m_per // 2, k), f8),
            pltpu.VMEM((m_per // 2, k), f8),
            pltpu.VMEM((m_per // 2, k), f8),
            pltpu.SemaphoreType.DMA,
            pltpu.SemaphoreType.DMA((2,)),
            pltpu.SemaphoreType.DMA((N_CW,)),
            pltpu.SemaphoreType.DMA((N_CW,)),
            pltpu.SemaphoreType.DMA((N_CW,)),
            pltpu.SemaphoreType.DMA((N_CW,)),
        ],
        compiler_params=pltpu.CompilerParams(
            collective_id=0, vmem_limit_bytes=60 * 1024 * 1024),
    )(x, w_mat, scale_x, scale_w)


# device time: 89788 ns/iter; 2.3156x vs baseline; 1.2808x over previous
import jax
import jax.numpy as jnp
from jax import lax
from jax.experimental import pallas as pl
from jax.experimental.pallas import tpu as pltpu

N_DEV = 8
W_PIECES = 4


def kernel(x, w_mat, scale_x, scale_w):
    m_per, k = x.shape
    n_total = w_mat.shape[1]
    n_per = n_total // N_DEV
    k_piece = k // W_PIECES
    m_half = m_per // 2

    def body(x_hbm, w_hbm, sx_ref, sw_ref, out_ref,
             x32_ref, x8_ref, w8_ref, wstage_ref,
             maj_ref, min_ref, sh_ref, majh_ref, shh_ref,
             minh_src, shh_src,
             xdma_sem, wdma_sems,
             maj_send, min_send, sh_send, maj_recv, min_recv, sh_recv):
        me = lax.axis_index("i").astype(jnp.int32)
        s = sx_ref[0] * sw_ref[0]

        idx = jnp.where(me < 4, me, 11 - me)
        sigma = jnp.int32(1) - 2 * lax.rem(idx, 2)

        def pos_at(j):
            jm = lax.rem(j + 2 * N_DEV, N_DEV)
            return jnp.where(jm < 4, jm, 11 - jm)

        d_maj = pos_at(idx + sigma)
        d_min = pos_at(idx - sigma)
        d_sh = pos_at(idx + 3 * sigma)
        col0 = pl.multiple_of(me * n_per, n_per)

        x_dma = pltpu.make_async_copy(x_hbm, x32_ref, xdma_sem)
        x_dma.start()

        def w_dma(j):
            return pltpu.make_async_copy(
                w_hbm.at[pl.ds(j * k_piece, k_piece), pl.ds(col0, n_per)],
                wstage_ref.at[j % 2],
                wdma_sems.at[j % 2],
            )

        w_dma(0).start()
        x_dma.wait()
        x8_ref[...] = x32_ref[...].astype(jnp.float8_e4m3fn)

        barrier_sem = pltpu.get_barrier_semaphore()
        for nbr in (d_maj, d_min, d_sh):
            pl.semaphore_signal(
                barrier_sem, inc=1,
                device_id=(nbr,), device_id_type=pl.DeviceIdType.MESH,
            )
        pl.semaphore_wait(barrier_sem, 3)

        def rdma(src, dst, snd, rcv, dev):
            return pltpu.make_async_remote_copy(
                src_ref=src, dst_ref=dst, send_sem=snd, recv_sem=rcv,
                device_id=(dev,), device_id_type=pl.DeviceIdType.MESH,
            )

        def M(h):
            src = x8_ref if h == 0 else maj_ref.at[0]
            return rdma(src, min_ref.at[h], maj_send.at[h], min_recv.at[h], d_maj)

        def N(h):
            if h == 2:
                return rdma(minh_src, majh_ref, min_send.at[2],
                            maj_recv.at[2], d_min)
            src = x8_ref if h == 0 else min_ref.at[0]
            return rdma(src, maj_ref.at[h], min_send.at[h], maj_recv.at[h], d_min)

        def S(h):
            if h == 2:
                return rdma(shh_src, shh_ref, sh_send.at[2],
                            sh_recv.at[2], d_sh)
            src = x8_ref if h == 0 else maj_ref.at[0]
            return rdma(src, sh_ref.at[h], sh_send.at[h], sh_recv.at[h], d_sh)

        M(0).start()
        N(0).start()
        S(0).start()

        for j in range(W_PIECES):
            if j + 1 < W_PIECES:
                w_dma(j + 1).start()
            w_dma(j).wait()
            w8_ref[pl.ds(j * k_piece, k_piece), :] = (
                wstage_ref[j % 2].astype(jnp.float8_e4m3fn))

        def chunk_gemm(src_ref, origin):
            acc = jnp.dot(src_ref[...], w8_ref[...],
                          preferred_element_type=jnp.float32)
            out_ref[pl.ds(origin * m_per, m_per), :] = jnp.maximum(acc * s, 0.0)

        chunk_gemm(x8_ref, me)

        N(0).wait_recv()
        M(1).start()
        S(1).start()
        M(0).wait_recv()
        N(1).start()
        chunk_gemm(maj_ref.at[0], pos_at(idx - sigma))
        S(0).wait_recv()
        chunk_gemm(min_ref.at[0], pos_at(idx + sigma))
        chunk_gemm(sh_ref.at[0], pos_at(idx + 3 * sigma))

        N(1).wait_recv()
        shh_src[...] = maj_ref[1, m_half:, :]
        S(2).start()
        M(1).wait_recv()
        minh_src[...] = min_ref[1, :m_half, :]
        N(2).start()
        chunk_gemm(maj_ref.at[1], pos_at(idx - 2 * sigma))
        S(1).wait_recv()
        chunk_gemm(min_ref.at[1], pos_at(idx + 2 * sigma))
        chunk_gemm(sh_ref.at[1], pos_at(idx + 4))

        o3 = pos_at(idx - 3 * sigma)
        N(2).wait_recv()
        acc_top = jnp.dot(majh_ref[...], w8_ref[...],
                          preferred_element_type=jnp.float32)
        out_ref[pl.ds(o3 * m_per, m_half), :] = jnp.maximum(acc_top * s, 0.0)
        S(2).wait_recv()
        acc_bot = jnp.dot(shh_ref[...], w8_ref[...],
                          preferred_element_type=jnp.float32)
        out_ref[pl.ds(o3 * m_per + m_half, m_half), :] = (
            jnp.maximum(acc_bot * s, 0.0))

        for h in range(2):
            M(h).wait_send()
        for h in range(3):
            N(h).wait_send()
            S(h).wait_send()

    f8 = jnp.float8_e4m3fn
    return pl.pallas_call(
        body,
        out_shape=jax.ShapeDtypeStruct((N_DEV * m_per, n_per), jnp.float32),
        in_specs=[
            pl.BlockSpec(memory_space=pl.ANY),
            pl.BlockSpec(memory_space=pl.ANY),
            pl.BlockSpec(memory_space=pltpu.SMEM),
            pl.BlockSpec(memory_space=pltpu.SMEM),
        ],
        out_specs=pl.BlockSpec(memory_space=pltpu.VMEM),
        scratch_shapes=[
            pltpu.VMEM((m_per, k), jnp.float32),
            pltpu.VMEM((m_per, k), f8),
            pltpu.VMEM((k, n_per), f8),
            pltpu.VMEM((2, k_piece, n_per), jnp.float32),
            pltpu.VMEM((2, m_per, k), f8),
            pltpu.VMEM((2, m_per, k), f8),
            pltpu.VMEM((2, m_per, k), f8),
            pltpu.VMEM((m_half, k), f8),
            pltpu.VMEM((m_half, k), f8),
            pltpu.VMEM((m_half, k), f8),
            pltpu.VMEM((m_half, k), f8),
            pltpu.SemaphoreType.DMA,
            pltpu.SemaphoreType.DMA((2,)),
            pltpu.SemaphoreType.DMA((2,)),
            pltpu.SemaphoreType.DMA((3,)),
            pltpu.SemaphoreType.DMA((3,)),
            pltpu.SemaphoreType.DMA((3,)),
            pltpu.SemaphoreType.DMA((2,)),
            pltpu.SemaphoreType.DMA((3,)),
        ],
        compiler_params=pltpu.CompilerParams(
            collective_id=0, vmem_limit_bytes=60 * 1024 * 1024),
    )(x, w_mat, scale_x, scale_w)


# device time: 87954 ns/iter; 2.3639x vs baseline; 1.0209x over previous
import jax
import jax.numpy as jnp
from jax import lax
from jax.experimental import pallas as pl
from jax.experimental.pallas import tpu as pltpu

N_DEV = 8
W_PIECES = 4


def kernel(x, w_mat, scale_x, scale_w):
    m_per, k = x.shape
    n_total = w_mat.shape[1]
    n_per = n_total // N_DEV
    k_piece = k // W_PIECES
    m_half = m_per // 2

    def body(x_hbm, w_hbm, sx_ref, sw_ref, out_ref,
             x32_ref, x8_ref, w8_ref, wstage_ref,
             majA, majB, minA, minB, shA, shB, majh, shh,
             xdma_sem, wdma_sems,
             maj_send, min_send, sh_send, maj_recv, min_recv, sh_recv):
        me = lax.axis_index("i").astype(jnp.int32)
        s = sx_ref[0] * sw_ref[0]

        idx = jnp.where(me < 4, me, 11 - me)
        sigma = jnp.int32(1) - 2 * lax.rem(idx, 2)

        def pos_at(j):
            jm = lax.rem(j + 2 * N_DEV, N_DEV)
            return jnp.where(jm < 4, jm, 11 - jm)

        d_maj = pos_at(idx + sigma)
        d_min = pos_at(idx - sigma)
        d_sh = pos_at(idx + 3 * sigma)
        col0 = pl.multiple_of(me * n_per, n_per)

        x_dma = pltpu.make_async_copy(x_hbm, x32_ref, xdma_sem)
        x_dma.start()

        def w_dma(j):
            return pltpu.make_async_copy(
                w_hbm.at[pl.ds(j * k_piece, k_piece), pl.ds(col0, n_per)],
                wstage_ref.at[j % 2],
                wdma_sems.at[j % 2],
            )

        w_dma(0).start()
        x_dma.wait()
        x8_ref[0] = x32_ref[:m_half, :].astype(jnp.float8_e4m3fn)
        x8_ref[1] = x32_ref[m_half:, :].astype(jnp.float8_e4m3fn)

        barrier_sem = pltpu.get_barrier_semaphore()
        for nbr in (d_maj, d_min, d_sh):
            pl.semaphore_signal(
                barrier_sem, inc=1,
                device_id=(nbr,), device_id_type=pl.DeviceIdType.MESH,
            )
        pl.semaphore_wait(barrier_sem, 3)

        def rdma(src, dst, snd, rcv, dev):
            return pltpu.make_async_remote_copy(
                src_ref=src, dst_ref=dst, send_sem=snd, recv_sem=rcv,
                device_id=(dev,), device_id_type=pl.DeviceIdType.MESH,
            )

        def MA(h):
            src = x8_ref.at[0] if h == 0 else majA.at[0]
            return rdma(src, minA.at[h], maj_send.at[h], min_recv.at[h], d_maj)

        def MB(h):
            src = x8_ref.at[1] if h == 0 else majB.at[0]
            return rdma(src, minB.at[h], maj_send.at[2 + h],
                        min_recv.at[2 + h], d_maj)

        def NA(h):
            src = x8_ref.at[0] if h == 0 else minA.at[0]
            return rdma(src, majA.at[h], min_send.at[h], maj_recv.at[h], d_min)

        def NB(h):
            src = x8_ref.at[1] if h == 0 else minB.at[0]
            return rdma(src, majB.at[h], min_send.at[2 + h],
                        maj_recv.at[2 + h], d_min)

        def NH():
            return rdma(minA.at[1], majh, min_send.at[4], maj_recv.at[4], d_min)

        def SA(h):
            src = x8_ref.at[0] if h == 0 else majA.at[0]
            return rdma(src, shA.at[h], sh_send.at[h], sh_recv.at[h], d_sh)

        def SB(h):
            src = x8_ref.at[1] if h == 0 else majB.at[0]
            return rdma(src, shB.at[h], sh_send.at[2 + h],
                        sh_recv.at[2 + h], d_sh)

        def SH():
            return rdma(majB.at[1], shh, sh_send.at[4], sh_recv.at[4], d_sh)

        MA(0).start()
        MB(0).start()
        NA(0).start()
        NB(0).start()
        SA(0).start()
        SB(0).start()

        for j in range(W_PIECES):
            if j + 1 < W_PIECES:
                w_dma(j + 1).start()
            w_dma(j).wait()
            w8_ref[pl.ds(j * k_piece, k_piece), :] = (
                wstage_ref[j % 2].astype(jnp.float8_e4m3fn))

        def half_gemm(src_ref, origin, bottom):
            acc = jnp.dot(src_ref[...], w8_ref[...],
                          preferred_element_type=jnp.float32)
            row0 = origin * m_per + (m_half if bottom else 0)
            out_ref[pl.ds(row0, m_half), :] = jnp.maximum(acc * s, 0.0)

        o_m1, o_m2, o_m3 = (pos_at(idx - sigma), pos_at(idx - 2 * sigma),
                            pos_at(idx - 3 * sigma))
        o_p1, o_p2, o_p3 = (pos_at(idx + sigma), pos_at(idx + 2 * sigma),
                            pos_at(idx + 3 * sigma))
        o_p4 = pos_at(idx + 4)

        half_gemm(x8_ref.at[0], me, False)
        half_gemm(x8_ref.at[1], me, True)

        NA(0).wait_recv()
        MA(1).start()
        SA(1).start()
        MA(0).wait_recv()
        NA(1).start()
        half_gemm(majA.at[0], o_m1, False)
        SA(0).wait_recv()
        half_gemm(minA.at[0], o_p1, False)
        half_gemm(shA.at[0], o_p3, False)

        NB(0).wait_recv()
        MB(1).start()
        SB(1).start()
        MB(0).wait_recv()
        NB(1).start()
        half_gemm(majB.at[0], o_m1, True)
        SB(0).wait_recv()
        half_gemm(minB.at[0], o_p1, True)
        half_gemm(shB.at[0], o_p3, True)

        MA(1).wait_recv()
        NH().start()
        NA(1).wait_recv()
        half_gemm(majA.at[1], o_m2, False)
        SA(1).wait_recv()
        half_gemm(minA.at[1], o_p2, False)
        half_gemm(shA.at[1], o_p4, False)

        NB(1).wait_recv()
        SH().start()
        MB(1).wait_recv()
        half_gemm(majB.at[1], o_m2, True)
        SB(1).wait_recv()
        half_gemm(minB.at[1], o_p2, True)
        half_gemm(shB.at[1], o_p4, True)

        NH().wait_recv()
        half_gemm(majh, o_m3, False)
        SH().wait_recv()
        half_gemm(shh, o_m3, True)

        for h in range(2):
            MA(h).wait_send()
            MB(h).wait_send()
            NA(h).wait_send()
            NB(h).wait_send()
            SA(h).wait_send()
            SB(h).wait_send()
        NH().wait_send()
        SH().wait_send()

    f8 = jnp.float8_e4m3fn
    half = (m_per // 2, k)
    return pl.pallas_call(
        body,
        out_shape=jax.ShapeDtypeStruct((N_DEV * m_per, n_per), jnp.float32),
        in_specs=[
            pl.BlockSpec(memory_space=pl.ANY),
            pl.BlockSpec(memory_space=pl.ANY),
            pl.BlockSpec(memory_space=pltpu.SMEM),
            pl.BlockSpec(memory_space=pltpu.SMEM),
        ],
        out_specs=pl.BlockSpec(memory_space=pltpu.VMEM),
        scratch_shapes=[
            pltpu.VMEM((m_per, k), jnp.float32),
            pltpu.VMEM((2,) + half, f8),
            pltpu.VMEM((k, n_per), f8),
            pltpu.VMEM((2, k_piece, n_per), jnp.float32),
            pltpu.VMEM((2,) + half, f8),
            pltpu.VMEM((2,) + half, f8),
            pltpu.VMEM((2,) + half, f8),
            pltpu.VMEM((2,) + half, f8),
            pltpu.VMEM((2,) + half, f8),
            pltpu.VMEM((2,) + half, f8),
            pltpu.VMEM(half, f8),
            pltpu.VMEM(half, f8),
            pltpu.SemaphoreType.DMA,
            pltpu.SemaphoreType.DMA((2,)),
            pltpu.SemaphoreType.DMA((4,)),
            pltpu.SemaphoreType.DMA((5,)),
            pltpu.SemaphoreType.DMA((5,)),
            pltpu.SemaphoreType.DMA((5,)),
            pltpu.SemaphoreType.DMA((4,)),
            pltpu.SemaphoreType.DMA((5,)),
        ],
        compiler_params=pltpu.CompilerParams(
            collective_id=0, vmem_limit_bytes=60 * 1024 * 1024),
    )(x, w_mat, scale_x, scale_w)


# device time: 86340 ns/iter; 2.4081x vs baseline; 1.0187x over previous
import jax
import jax.numpy as jnp
from jax import lax
from jax.experimental import pallas as pl
from jax.experimental.pallas import tpu as pltpu

N_DEV = 8
W_PIECES = 4


def kernel(x, w_mat, scale_x, scale_w):
    m_per, k = x.shape
    n_total = w_mat.shape[1]
    n_per = n_total // N_DEV
    k_piece = k // W_PIECES
    m_half = m_per // 2

    def body(x_hbm, w_hbm, sx_ref, sw_ref, out_ref,
             x32_ref, x8_ref, w8_ref, wstage_ref,
             majA, majB, minA, minB, shA, shB, majh, shh,
             xdma_sem, wdma_sems,
             maj_send, min_send, sh_send, maj_recv, min_recv, sh_recv):
        me = lax.axis_index("i").astype(jnp.int32)
        s = sx_ref[0] * sw_ref[0]

        idx = jnp.where(me < 4, me, 11 - me)
        sigma = jnp.int32(1) - 2 * lax.rem(idx, 2)

        def pos_at(j):
            jm = lax.rem(j + 2 * N_DEV, N_DEV)
            return jnp.where(jm < 4, jm, 11 - jm)

        d_maj = pos_at(idx + sigma)
        d_min = pos_at(idx - sigma)
        d_sh = pos_at(idx + 3 * sigma)
        col0 = pl.multiple_of(me * n_per, n_per)

        x_dma_a = pltpu.make_async_copy(
            x_hbm.at[pl.ds(0, m_half), :], x32_ref.at[pl.ds(0, m_half), :],
            xdma_sem.at[0])
        x_dma_b = pltpu.make_async_copy(
            x_hbm.at[pl.ds(m_half, m_half), :],
            x32_ref.at[pl.ds(m_half, m_half), :], xdma_sem.at[1])
        x_dma_a.start()
        x_dma_b.start()

        def w_dma(j):
            return pltpu.make_async_copy(
                w_hbm.at[pl.ds(j * k_piece, k_piece), pl.ds(col0, n_per)],
                wstage_ref.at[j % 2],
                wdma_sems.at[j % 2],
            )

        w_dma(0).start()
        x_dma_a.wait()
        x8_ref[0] = x32_ref[:m_half, :].astype(jnp.float8_e4m3fn)

        barrier_sem = pltpu.get_barrier_semaphore()
        for nbr in (d_maj, d_min, d_sh):
            pl.semaphore_signal(
                barrier_sem, inc=1,
                device_id=(nbr,), device_id_type=pl.DeviceIdType.MESH,
            )
        pl.semaphore_wait(barrier_sem, 3)

        def rdma(src, dst, snd, rcv, dev):
            return pltpu.make_async_remote_copy(
                src_ref=src, dst_ref=dst, send_sem=snd, recv_sem=rcv,
                device_id=(dev,), device_id_type=pl.DeviceIdType.MESH,
            )

        def MA(h):
            src = x8_ref.at[0] if h == 0 else majA.at[0]
            return rdma(src, minA.at[h], maj_send.at[h], min_recv.at[h], d_maj)

        def MB(h):
            src = x8_ref.at[1] if h == 0 else majB.at[0]
            return rdma(src, minB.at[h], maj_send.at[2 + h],
                        min_recv.at[2 + h], d_maj)

        def NA(h):
            src = x8_ref.at[0] if h == 0 else minA.at[0]
            return rdma(src, majA.at[h], min_send.at[h], maj_recv.at[h], d_min)

        def NB(h):
            src = x8_ref.at[1] if h == 0 else minB.at[0]
            return rdma(src, majB.at[h], min_send.at[2 + h],
                        maj_recv.at[2 + h], d_min)

        def NH():
            return rdma(minA.at[1], majh, min_send.at[4], maj_recv.at[4], d_min)

        def SA(h):
            src = x8_ref.at[0] if h == 0 else majA.at[0]
            return rdma(src, shA.at[h], sh_send.at[h], sh_recv.at[h], d_sh)

        def SB(h):
            src = x8_ref.at[1] if h == 0 else majB.at[0]
            return rdma(src, shB.at[h], sh_send.at[2 + h],
                        sh_recv.at[2 + h], d_sh)

        def SH():
            return rdma(majB.at[1], shh, sh_send.at[4], sh_recv.at[4], d_sh)

        MA(0).start()
        NA(0).start()
        SA(0).start()
        x_dma_b.wait()
        x8_ref[1] = x32_ref[m_half:, :].astype(jnp.float8_e4m3fn)
        MB(0).start()
        NB(0).start()
        SB(0).start()

        for j in range(W_PIECES):
            if j + 1 < W_PIECES:
                w_dma(j + 1).start()
            w_dma(j).wait()
            w8_ref[pl.ds(j * k_piece, k_piece), :] = (
                wstage_ref[j % 2].astype(jnp.float8_e4m3fn))

        def half_gemm(src_ref, origin, bottom):
            acc = jnp.dot(src_ref[...], w8_ref[...],
                          preferred_element_type=jnp.float32)
            row0 = origin * m_per + (m_half if bottom else 0)
            out_ref[pl.ds(row0, m_half), :] = jnp.maximum(acc * s, 0.0)

        o_m1, o_m2, o_m3 = (pos_at(idx - sigma), pos_at(idx - 2 * sigma),
                            pos_at(idx - 3 * sigma))
        o_p1, o_p2, o_p3 = (pos_at(idx + sigma), pos_at(idx + 2 * sigma),
                            pos_at(idx + 3 * sigma))
        o_p4 = pos_at(idx + 4)

        half_gemm(x8_ref.at[0], me, False)
        half_gemm(x8_ref.at[1], me, True)

        NA(0).wait_recv()
        MA(1).start()
        SA(1).start()
        MA(0).wait_recv()
        NA(1).start()
        half_gemm(majA.at[0], o_m1, False)
        SA(0).wait_recv()
        half_gemm(minA.at[0], o_p1, False)
        half_gemm(shA.at[0], o_p3, False)

        NB(0).wait_recv()
        MB(1).start()
        SB(1).start()
        MB(0).wait_recv()
        NB(1).start()
        half_gemm(majB.at[0], o_m1, True)
        SB(0).wait_recv()
        half_gemm(minB.at[0], o_p1, True)
        half_gemm(shB.at[0], o_p3, True)

        MA(1).wait_recv()
        NH().start()
        NA(1).wait_recv()
        half_gemm(majA.at[1], o_m2, False)
        SA(1).wait_recv()
        half_gemm(minA.at[1], o_p2, False)
        half_gemm(shA.at[1], o_p4, False)

        NB(1).wait_recv()
        SH().start()
        MB(1).wait_recv()
        half_gemm(majB.at[1], o_m2, True)
        SB(1).wait_recv()
        half_gemm(minB.at[1], o_p2, True)
        half_gemm(shB.at[1], o_p4, True)

        NH().wait_recv()
        half_gemm(majh, o_m3, False)
        SH().wait_recv()
        half_gemm(shh, o_m3, True)

        for h in range(2):
            MA(h).wait_send()
            MB(h).wait_send()
            NA(h).wait_send()
            NB(h).wait_send()
            SA(h).wait_send()
            SB(h).wait_send()
        NH().wait_send()
        SH().wait_send()

    f8 = jnp.float8_e4m3fn
    half = (m_per // 2, k)
    return pl.pallas_call(
        body,
        out_shape=jax.ShapeDtypeStruct((N_DEV * m_per, n_per), jnp.float32),
        in_specs=[
            pl.BlockSpec(memory_space=pl.ANY),
            pl.BlockSpec(memory_space=pl.ANY),
            pl.BlockSpec(memory_space=pltpu.SMEM),
            pl.BlockSpec(memory_space=pltpu.SMEM),
        ],
        out_specs=pl.BlockSpec(memory_space=pltpu.VMEM),
        scratch_shapes=[
            pltpu.VMEM((m_per, k), jnp.float32),
            pltpu.VMEM((2,) + half, f8),
            pltpu.VMEM((k, n_per), f8),
            pltpu.VMEM((2, k_piece, n_per), jnp.float32),
            pltpu.VMEM((2,) + half, f8),
            pltpu.VMEM((2,) + half, f8),
            pltpu.VMEM((2,) + half, f8),
            pltpu.VMEM((2,) + half, f8),
            pltpu.VMEM((2,) + half, f8),
            pltpu.VMEM((2,) + half, f8),
            pltpu.VMEM(half, f8),
            pltpu.VMEM(half, f8),
            pltpu.SemaphoreType.DMA((2,)),
            pltpu.SemaphoreType.DMA((2,)),
            pltpu.SemaphoreType.DMA((4,)),
            pltpu.SemaphoreType.DMA((5,)),
            pltpu.SemaphoreType.DMA((5,)),
            pltpu.SemaphoreType.DMA((5,)),
            pltpu.SemaphoreType.DMA((4,)),
            pltpu.SemaphoreType.DMA((5,)),
        ],
        compiler_params=pltpu.CompilerParams(
            collective_id=0, vmem_limit_bytes=60 * 1024 * 1024),
    )(x, w_mat, scale_x, scale_w)


# device time: 86252 ns/iter; 2.4105x vs baseline; 1.0010x over previous
import jax
import jax.numpy as jnp
from jax import lax
from jax.experimental import pallas as pl
from jax.experimental.pallas import tpu as pltpu

N_DEV = 8
W_PIECES = 4


def kernel(x, w_mat, scale_x, scale_w):
    m_per, k = x.shape
    n_total = w_mat.shape[1]
    n_per = n_total // N_DEV
    k_piece = k // W_PIECES
    m_half = m_per // 2

    def body(x_hbm, w_hbm, sx_ref, sw_ref, out_ref,
             x32_ref, x8_ref, w8_ref, wstage_ref,
             majA, majB, minA, minB, shA, shB, majh, shh,
             xdma_sem, wdma_sems,
             maj_send, min_send, sh_send, maj_recv, min_recv, sh_recv):
        me = lax.axis_index("i").astype(jnp.int32)
        s = sx_ref[0] * sw_ref[0]

        idx = jnp.where(me < 4, me, 11 - me)
        sigma = jnp.int32(1) - 2 * lax.rem(idx, 2)

        def pos_at(j):
            jm = lax.rem(j + 2 * N_DEV, N_DEV)
            return jnp.where(jm < 4, jm, 11 - jm)

        d_maj = pos_at(idx + sigma)
        d_min = pos_at(idx - sigma)
        d_sh = pos_at(idx + 3 * sigma)
        col0 = pl.multiple_of(me * n_per, n_per)

        x_dma_a = pltpu.make_async_copy(
            x_hbm.at[pl.ds(0, m_half), :], x32_ref.at[pl.ds(0, m_half), :],
            xdma_sem.at[0])
        x_dma_b = pltpu.make_async_copy(
            x_hbm.at[pl.ds(m_half, m_half), :],
            x32_ref.at[pl.ds(m_half, m_half), :], xdma_sem.at[1])
        x_dma_a.start()
        x_dma_b.start()

        def w_dma(j):
            return pltpu.make_async_copy(
                w_hbm.at[pl.ds(j * k_piece, k_piece), pl.ds(col0, n_per)],
                wstage_ref.at[j % 2],
                wdma_sems.at[j % 2],
            )

        w_dma(0).start()
        x_dma_a.wait()
        x8_ref[0] = x32_ref[:m_half, :].astype(jnp.float8_e4m3fn)

        barrier_sem = pltpu.get_barrier_semaphore()
        for nbr in (d_maj, d_min, d_sh):
            pl.semaphore_signal(
                barrier_sem, inc=1,
                device_id=(nbr,), device_id_type=pl.DeviceIdType.MESH,
            )
        pl.semaphore_wait(barrier_sem, 3)

        def rdma(src, dst, snd, rcv, dev):
            return pltpu.make_async_remote_copy(
                src_ref=src, dst_ref=dst, send_sem=snd, recv_sem=rcv,
                device_id=(dev,), device_id_type=pl.DeviceIdType.MESH,
            )

        def MA(h):
            src = x8_ref.at[0] if h == 0 else majA.at[0]
            return rdma(src, minA.at[h], maj_send.at[h], min_recv.at[h], d_maj)

        def MB(h):
            src = x8_ref.at[1] if h == 0 else majB.at[0]
            return rdma(src, minB.at[h], maj_send.at[2 + h],
                        min_recv.at[2 + h], d_maj)

        def NA(h):
            src = x8_ref.at[0] if h == 0 else minA.at[0]
            return rdma(src, majA.at[h], min_send.at[h], maj_recv.at[h], d_min)

        def NB(h):
            src = x8_ref.at[1] if h == 0 else minB.at[0]
            return rdma(src, majB.at[h], min_send.at[2 + h],
                        maj_recv.at[2 + h], d_min)

        def NH():
            return rdma(minA.at[1], majh, min_send.at[4], maj_recv.at[4], d_min)

        def SA(h):
            src = x8_ref.at[0] if h == 0 else majA.at[0]
            return rdma(src, shA.at[h], sh_send.at[h], sh_recv.at[h], d_sh)

        def SB(h):
            src = x8_ref.at[1] if h == 0 else majB.at[0]
            return rdma(src, shB.at[h], sh_send.at[2 + h],
                        sh_recv.at[2 + h], d_sh)

        def SH():
            return rdma(majB.at[1], shh, sh_send.at[4], sh_recv.at[4], d_sh)

        MA(0).start()
        NA(0).start()
        SA(0).start()
        x_dma_b.wait()
        x8_ref[1] = x32_ref[m_half:, :].astype(jnp.float8_e4m3fn)
        MB(0).start()
        NB(0).start()
        SB(0).start()

        for j in range(W_PIECES):
            if j + 1 < W_PIECES:
                w_dma(j + 1).start()
            w_dma(j).wait()
            w8_ref[pl.ds(j * k_piece, k_piece), :] = (
                wstage_ref[j % 2].astype(jnp.float8_e4m3fn))

        def half_gemm(src_ref, origin, bottom):
            acc = jnp.dot(src_ref[...], w8_ref[...],
                          preferred_element_type=jnp.float32)
            row0 = origin * m_per + (m_half if bottom else 0)
            out_ref[pl.ds(row0, m_half), :] = jnp.maximum(acc * s, 0.0)

        o_m1, o_m2, o_m3 = (pos_at(idx - sigma), pos_at(idx - 2 * sigma),
                            pos_at(idx - 3 * sigma))
        o_p1, o_p2, o_p3 = (pos_at(idx + sigma), pos_at(idx + 2 * sigma),
                            pos_at(idx + 3 * sigma))
        o_p4 = pos_at(idx + 4)

        NA(0).wait_recv()
        MA(1).start()
        SA(1).start()
        MA(0).wait_recv()
        NA(1).start()
        half_gemm(x8_ref.at[0], me, False)
        half_gemm(x8_ref.at[1], me, True)
        NB(0).wait_recv()
        MB(1).start()
        SB(1).start()
        MB(0).wait_recv()
        NB(1).start()
        SA(0).wait_recv()
        half_gemm(majA.at[0], o_m1, False)
        half_gemm(minA.at[0], o_p1, False)
        half_gemm(shA.at[0], o_p3, False)
        MA(1).wait_recv()
        NH().start()
        SB(0).wait_recv()
        half_gemm(majB.at[0], o_m1, True)
        half_gemm(minB.at[0], o_p1, True)
        half_gemm(shB.at[0], o_p3, True)
        half_gemm(minA.at[1], o_p2, False)
        NB(1).wait_recv()
        SH().start()
        NA(1).wait_recv()
        SA(1).wait_recv()
        half_gemm(majA.at[1], o_m2, False)
        half_gemm(shA.at[1], o_p4, False)
        MB(1).wait_recv()
        SB(1).wait_recv()
        half_gemm(majB.at[1], o_m2, True)
        half_gemm(minB.at[1], o_p2, True)
        half_gemm(shB.at[1], o_p4, True)
        NH().wait_recv()
        half_gemm(majh, o_m3, False)
        SH().wait_recv()
        half_gemm(shh, o_m3, True)

        for h in range(2):
            MA(h).wait_send()
            MB(h).wait_send()
            NA(h).wait_send()
            NB(h).wait_send()
            SA(h).wait_send()
            SB(h).wait_send()
        NH().wait_send()
        SH().wait_send()

    f8 = jnp.float8_e4m3fn
    half = (m_per // 2, k)
    return pl.pallas_call(
        body,
        out_shape=jax.ShapeDtypeStruct((N_DEV * m_per, n_per), jnp.float32),
        in_specs=[
            pl.BlockSpec(memory_space=pl.ANY),
            pl.BlockSpec(memory_space=pl.ANY),
            pl.BlockSpec(memory_space=pltpu.SMEM),
            pl.BlockSpec(memory_space=pltpu.SMEM),
        ],
        out_specs=pl.BlockSpec(memory_space=pltpu.VMEM),
        scratch_shapes=[
            pltpu.VMEM((m_per, k), jnp.float32),
            pltpu.VMEM((2,) + half, f8),
            pltpu.VMEM((k, n_per), f8),
            pltpu.VMEM((2, k_piece, n_per), jnp.float32),
            pltpu.VMEM((2,) + half, f8),
            pltpu.VMEM((2,) + half, f8),
            pltpu.VMEM((2,) + half, f8),
            pltpu.VMEM((2,) + half, f8),
            pltpu.VMEM((2,) + half, f8),
            pltpu.VMEM((2,) + half, f8),
            pltpu.VMEM(half, f8),
            pltpu.VMEM(half, f8),
            pltpu.SemaphoreType.DMA((2,)),
            pltpu.SemaphoreType.DMA((2,)),
            pltpu.SemaphoreType.DMA((4,)),
            pltpu.SemaphoreType.DMA((5,)),
            pltpu.SemaphoreType.DMA((5,)),
            pltpu.SemaphoreType.DMA((5,)),
            pltpu.SemaphoreType.DMA((4,)),
            pltpu.SemaphoreType.DMA((5,)),
        ],
        compiler_params=pltpu.CompilerParams(
            collective_id=0, vmem_limit_bytes=60 * 1024 * 1024),
    )(x, w_mat, scale_x, scale_w)
